# Initial kernel scaffold; baseline (speedup 1.0000x reference)
import jax
import jax.numpy as jnp
from jax.experimental import pallas as pl


def kernel(x, emb_weight):
    raise NotImplementedError("write your pallas kernel here")


# speedup vs baseline: 4.3749x; 4.3749x over previous
import functools

import jax
import jax.numpy as jnp
from jax import lax
from jax.experimental import pallas as pl
from jax.experimental.pallas import tpu as pltpu
from jax.experimental.pallas import tpu_sc as plsc

_B, _S, _D = 4096, 50, 64
_N = _B * _S
_NC, _NS = 2, 16
_NW = _NC * _NS
_PER_W = _N // _NW
_CHUNK = 400
_NCHUNKS = _PER_W // _CHUNK


def kernel(x, emb_weight):
    idx = x.reshape(_N).astype(jnp.int32)

    @functools.partial(
        pl.kernel,
        out_type=jax.ShapeDtypeStruct((_N, _D), emb_weight.dtype),
        mesh=plsc.VectorSubcoreMesh(core_axis_name="c", subcore_axis_name="s"),
        scratch_types=[
            pltpu.VMEM((_CHUNK,), jnp.int32),
            pltpu.VMEM((_CHUNK, _D), jnp.float32),
            pltpu.SemaphoreType.DMA,
        ],
        compiler_params=pltpu.CompilerParams(use_tc_tiling_on_sc=False),
    )
    def gather_kernel(table_hbm, idx_hbm, out_hbm, idx_v, rows_v, sem):
        wid = lax.axis_index("s") * _NC + lax.axis_index("c")
        base = wid * _PER_W

        @pl.loop(0, _NCHUNKS)
        def _(c):
            row0 = base + c * _CHUNK
            pltpu.sync_copy(idx_hbm.at[pl.ds(row0, _CHUNK)], idx_v)
            pltpu.async_copy(table_hbm.at[idx_v], rows_v, sem).wait()
            pltpu.sync_copy(rows_v, out_hbm.at[pl.ds(row0, _CHUNK)])

    return gather_kernel(emb_weight, idx).reshape(_B, _S, _D)


# speedup vs baseline: 4.5998x; 1.0514x over previous
import functools

import jax
import jax.numpy as jnp
from jax.experimental import pallas as pl
from jax.experimental.pallas import tpu as pltpu
from jax.experimental.pallas import tpu_sc as plsc

_B, _S, _D = 4096, 50, 64
_N = _B * _S
_WINDOW = 512


def kernel(x, emb_weight):
    idx = x.reshape(1, _N).astype(jnp.int32)

    @functools.partial(
        pl.kernel,
        out_type=jax.ShapeDtypeStruct((_N, _D), emb_weight.dtype),
        mesh=plsc.VectorSubcoreMesh(core_axis_name="c", subcore_axis_name="s"),
        compiler_params=pltpu.CompilerParams(use_tc_tiling_on_sc=False),
    )
    def gather_kernel(table_hbm, idx_hbm, out_hbm):
        def body(idx_vmem, out_vmem):
            pltpu.sync_copy(table_hbm.at[idx_vmem.at[0]], out_vmem)

        pltpu.emit_pipeline(
            body,
            grid=(_N // _WINDOW,),
            in_specs=[pl.BlockSpec((1, _WINDOW), index_map=lambda i: (0, i))],
            out_specs=[pl.BlockSpec((_WINDOW, _D), index_map=lambda i: (i, 0))],
            core_axis_name=("c", "s"),
            dimension_semantics=(pltpu.PARALLEL,),
        )(idx_hbm, out_hbm)

    return gather_kernel(emb_weight, idx).reshape(_B, _S, _D)
